# Initial kernel scaffold; baseline (speedup 1.0000x reference)
import jax
import jax.numpy as jnp
from jax.experimental import pallas as pl


def kernel(x, patch_ids, ln_scale, ln_bias, W_qkv, b_qkv, W_proj, b_proj):
    raise NotImplementedError("write your pallas kernel here")


# speedup vs baseline: 1.1817x; 1.1817x over previous
import functools

import jax
import jax.numpy as jnp
from jax.experimental import pallas as pl
from jax.experimental.pallas import tpu as pltpu

N = 2048
C = 512
H = 8
DH = C // H
NQ = 4
QB = N // NQ
NEG = -1e9


def _attn_kernel(x_ref, ids_row_ref, ids_col_ref, lns_ref, lnb_ref,
                 w8_ref, b8_ref, p8_ref, bproj_ref, out_ref,
                 xn_bf, qkv_s, bias_s):
    h = pl.program_id(0)
    qb = pl.program_id(1)

    @pl.when((h == 0) & (qb == 0))
    def _prologue():
        x = x_ref[...]
        mean = jnp.mean(x, axis=1, keepdims=True)
        xc = x - mean
        var = jnp.mean(xc * xc, axis=1, keepdims=True)
        xn = xc * jax.lax.rsqrt(var + 1e-5)
        xn = xn * lns_ref[...] + lnb_ref[...]
        xn_bf[...] = xn.astype(jnp.bfloat16)
        same = ids_col_ref[...] == ids_row_ref[...]
        bias_s[...] = jnp.where(same, 0.0, NEG).astype(jnp.bfloat16)

    @pl.when(qb == 0)
    def _qkv():
        qkv = jax.lax.dot_general(
            xn_bf[...], w8_ref[0],
            (((1,), (1,)), ((), ())),
            preferred_element_type=jnp.float32)
        qkv = qkv + b8_ref[0]
        qkv_s[...] = qkv.astype(jnp.bfloat16)

    rows = pl.ds(qb * QB, QB)
    q = qkv_s[rows, 0:DH]
    k = qkv_s[:, DH:2 * DH]
    v = qkv_s[:, 2 * DH:3 * DH]
    logits = jax.lax.dot_general(
        q, k, (((1,), (1,)), ((), ())),
        preferred_element_type=jnp.float32) * (1.0 / (DH ** 0.5))
    logits = logits + bias_s[rows, :].astype(jnp.float32)
    m = jnp.max(logits, axis=1, keepdims=True)
    p = jnp.exp(logits - m)
    s = jnp.sum(p, axis=1, keepdims=True)
    y = jax.lax.dot_general(
        p.astype(jnp.bfloat16), v, (((1,), (0,)), ((), ())),
        preferred_element_type=jnp.float32)
    y = y * (1.0 / s)
    contrib = jax.lax.dot_general(
        y.astype(jnp.bfloat16), p8_ref[0], (((1,), (0,)), ((), ())),
        preferred_element_type=jnp.float32)

    @pl.when(h == 0)
    def _init():
        out_ref[rows, :] = x_ref[rows, :] + bproj_ref[...] + contrib

    @pl.when(h != 0)
    def _acc():
        out_ref[rows, :] += contrib


@jax.jit
def kernel(x, patch_ids, ln_scale, ln_bias, W_qkv, b_qkv, W_proj, b_proj):
    ids_f = patch_ids.astype(jnp.float32)
    ids_row = ids_f.reshape(1, N)
    ids_col = ids_f.reshape(N, 1)
    w8 = W_qkv.reshape(3, H, DH, C).transpose(1, 0, 2, 3).reshape(
        H, 3 * DH, C).astype(jnp.bfloat16)
    b8 = b_qkv.reshape(3, H, DH).transpose(1, 0, 2).reshape(H, 1, 3 * DH)
    p8 = W_proj.reshape(C, H, DH).transpose(1, 2, 0).astype(jnp.bfloat16)

    grid = (H, NQ)
    return pl.pallas_call(
        _attn_kernel,
        grid=grid,
        in_specs=[
            pl.BlockSpec((N, C), lambda h, qb: (0, 0)),
            pl.BlockSpec((1, N), lambda h, qb: (0, 0)),
            pl.BlockSpec((N, 1), lambda h, qb: (0, 0)),
            pl.BlockSpec((1, C), lambda h, qb: (0, 0)),
            pl.BlockSpec((1, C), lambda h, qb: (0, 0)),
            pl.BlockSpec((1, 3 * DH, C), lambda h, qb: (h, 0, 0)),
            pl.BlockSpec((1, 1, 3 * DH), lambda h, qb: (h, 0, 0)),
            pl.BlockSpec((1, DH, C), lambda h, qb: (h, 0, 0)),
            pl.BlockSpec((1, C), lambda h, qb: (0, 0)),
        ],
        out_specs=pl.BlockSpec((N, C), lambda h, qb: (0, 0)),
        out_shape=jax.ShapeDtypeStruct((N, C), jnp.float32),
        scratch_shapes=[
            pltpu.VMEM((N, C), jnp.bfloat16),
            pltpu.VMEM((N, 3 * DH), jnp.bfloat16),
            pltpu.VMEM((N, N), jnp.bfloat16),
        ],
        compiler_params=pltpu.CompilerParams(
            dimension_semantics=("arbitrary", "arbitrary")),
    )(x, ids_row, ids_col, ln_scale.reshape(1, C), ln_bias.reshape(1, C),
      w8, b8, p8, b_proj.reshape(1, C))
